# baseline (speedup 1.0000x reference)
import functools
import math

import jax
import jax.numpy as jnp
from jax import lax
from jax.experimental import pallas as pl
from jax.experimental.pallas import tpu as pltpu, tpu_sc as plsc

B = 1024
L = 200
D = 128
SCALE = math.sqrt(float(D))

_NC = 2
_NS = 16
_NW = _NC * _NS
_Q = 4
_BQ = B // _Q
_NBLK = L * _Q // _NW
_NBUF = 3
_CH = 32
_NCH = _BQ // _CH
_NSP = 4
_CPW = B // _NW


def _sc_body(w_hbm, xq_hbm, pos_hbm, ct_hbm, out_hbm,
             ct_v, cls_v, posr_v, idx_v, obuf_v, spm_v, isem, gsem, osem,
             csem):
    sid = lax.axis_index("s")
    wid = sid * _NC + lax.axis_index("c")

    def prefetch(k):
        g = wid * _NBLK + k
        r1 = g // _Q
        q = g % _Q
        a = pltpu.make_async_copy(xq_hbm.at[r1, pl.ds(2 * q, 2)],
                                  idx_v.at[k], isem)
        b = pltpu.make_async_copy(pos_hbm.at[r1 + 1], posr_v.at[k], isem)
        return a, b

    for k in range(_NBLK):
        a, b = prefetch(k)
        a.start()
        b.start()

    pltpu.sync_copy(pos_hbm.at[0], cls_v)
    pltpu.sync_copy(ct_hbm, ct_v)
    for c in range(D // 16):
        sl = pl.ds(c * 16, 16)
        cls_v[sl] = cls_v[sl] + ct_v[sl]
    for b in range(_CPW):
        for c in range(D // 16):
            sl = pl.ds(c * 16, 16)
            obuf_v[0, b, sl] = cls_v[sl]
    pltpu.sync_copy(obuf_v.at[0, pl.ds(0, _CPW)],
                    out_hbm.at[0, pl.ds(wid * _CPW, _CPW)])

    for k in range(_NBLK):
        a, b = prefetch(k)
        a.wait()
        b.wait()

    def start_gather(k):
        s = k % _NBUF
        for j in range(2):
            pltpu.async_copy(w_hbm.at[idx_v.at[k, j]],
                             obuf_v.at[s, pl.ds(j * 128, 128)], gsem)

    def wait_gather(k):
        s = k % _NBUF
        for j in range(2):
            pltpu.make_async_copy(w_hbm.at[idx_v.at[k, j]],
                                  obuf_v.at[s, pl.ds(j * 128, 128)],
                                  gsem).wait()

    def out_dst(k):
        g = wid * _NBLK + k
        return out_hbm.at[g // _Q + 1, pl.ds((g % _Q) * _BQ, _BQ)]

    def xbar_desc(m):
        k, h = m // _NCH, m % _NCH
        return pltpu.make_async_copy(
            obuf_v.at[k % _NBUF, pl.ds(h * _CH, _CH)],
            spm_v.at[sid, m % _NSP], csem)

    def hbmw_desc(m):
        k, h = m // _NCH, m % _NCH
        g = wid * _NBLK + k
        dst = out_hbm.at[g // _Q + 1, pl.ds((g % _Q) * _BQ + h * _CH, _CH)]
        return pltpu.make_async_copy(spm_v.at[sid, m % _NSP], dst, osem)

    for k in range(min(2, _NBLK)):
        start_gather(k)

    for k in range(_NBLK):
        s = k % _NBUF
        wait_gather(k)

        tvals = [posr_v[k, pl.ds(c * 16, 16)] for c in range(D // 16)]

        def row_body(r, c2, s=s, tvals=tvals):
            for dr in range(2):
                row = 2 * r + dr
                for c in range(D // 16):
                    sl = pl.ds(c * 16, 16)
                    obuf_v[s, row, sl] = (obuf_v[s, row, sl] * SCALE
                                          + tvals[c])
            return c2

        lax.fori_loop(0, _BQ // 2, row_body, 0)

        for h in range(_NCH):
            m = k * _NCH + h
            if m >= _NSP:
                hbmw_desc(m - _NSP).wait()
            xbar_desc(m).start()
            if m >= 1:
                xbar_desc(m - 1).wait()
                hbmw_desc(m - 1).start()
            if h == 0 and k + 2 < _NBLK:
                start_gather(k + 2)

    _M = _NBLK * _NCH - 1
    xbar_desc(_M).wait()
    hbmw_desc(_M).start()
    for m in range(_M - _NSP + 1, _M + 1):
        hbmw_desc(m).wait()


@functools.partial(jax.jit, static_argnames=())
def kernel(x, W, class_token, pos_encoding):
    xq = x.astype(jnp.int32).T.reshape(L, B // 128, 128)
    pos = pos_encoding[0, : L + 1]
    ct = class_token.reshape(D)

    mesh = plsc.VectorSubcoreMesh(core_axis_name="c", subcore_axis_name="s")
    f = functools.partial(
        pl.kernel,
        mesh=mesh,
        out_type=jax.ShapeDtypeStruct((L + 1, B, D), jnp.float32),
        scratch_types=[
            pltpu.VMEM((D,), jnp.float32),
            pltpu.VMEM((D,), jnp.float32),
            pltpu.VMEM((_NBLK, D), jnp.float32),
            pltpu.VMEM((_NBLK, 2, 128), jnp.int32),
            pltpu.VMEM((_NBUF, _BQ, D), jnp.float32),
            pltpu.VMEM_SHARED((_NS, _NSP, _CH, D), jnp.float32),
            pltpu.SemaphoreType.DMA,
            pltpu.SemaphoreType.DMA,
            pltpu.SemaphoreType.DMA,
            pltpu.SemaphoreType.DMA,
        ],
    )(_sc_body)
    out = f(W, xq, pos, ct)
    return jnp.transpose(out, (1, 0, 2))


# speedup vs baseline: 1.0633x; 1.0633x over previous
import functools
import math

import jax
import jax.numpy as jnp
from jax import lax
from jax.experimental import pallas as pl
from jax.experimental.pallas import tpu as pltpu, tpu_sc as plsc

B = 1024
L = 200
D = 128
SCALE = math.sqrt(float(D))

_NC = 2
_NS = 16
_NW = _NC * _NS
_Q = 4
_BQ = B // _Q
_NBLK = L * _Q // _NW
_NBUF = 3
_CPW = B // _NW


def _sc_body(w_hbm, xq_hbm, pos_hbm, ct_hbm, out_hbm,
             ct_v, cls_v, posr_v, idx_v, obuf_v, isem, gsem, osem):
    wid = lax.axis_index("s") * _NC + lax.axis_index("c")

    def prefetch(k):
        g = wid * _NBLK + k
        r1 = g // _Q
        q = g % _Q
        a = pltpu.make_async_copy(xq_hbm.at[r1, pl.ds(2 * q, 2)],
                                  idx_v.at[k], isem)
        b = pltpu.make_async_copy(pos_hbm.at[r1 + 1], posr_v.at[k], isem)
        return a, b

    for k in range(_NBLK):
        a, b = prefetch(k)
        a.start()
        b.start()

    pltpu.sync_copy(pos_hbm.at[0], cls_v)
    pltpu.sync_copy(ct_hbm, ct_v)
    for c in range(D // 16):
        sl = pl.ds(c * 16, 16)
        cls_v[sl] = cls_v[sl] + ct_v[sl]
    for b in range(_CPW):
        for c in range(D // 16):
            sl = pl.ds(c * 16, 16)
            obuf_v[0, b, sl] = cls_v[sl]
    pltpu.sync_copy(obuf_v.at[0, pl.ds(0, _CPW)],
                    out_hbm.at[0, pl.ds(wid * _CPW, _CPW)])

    for k in range(_NBLK):
        a, b = prefetch(k)
        a.wait()
        b.wait()

    def start_gather(k):
        s = k % _NBUF
        for j in range(2):
            pltpu.async_copy(w_hbm.at[idx_v.at[k, j]],
                             obuf_v.at[s, pl.ds(j * 128, 128)], gsem)

    def wait_gather(k):
        s = k % _NBUF
        for j in range(2):
            pltpu.make_async_copy(w_hbm.at[idx_v.at[k, j]],
                                  obuf_v.at[s, pl.ds(j * 128, 128)],
                                  gsem).wait()

    def out_dst(k):
        g = wid * _NBLK + k
        return out_hbm.at[g // _Q + 1, pl.ds((g % _Q) * _BQ, _BQ)]

    def wait_write(k):
        pltpu.make_async_copy(obuf_v.at[k % _NBUF], out_dst(k), osem).wait()

    for k in range(min(2, _NBLK)):
        start_gather(k)

    for k in range(_NBLK):
        s = k % _NBUF
        wait_gather(k)

        tvals = [posr_v[k, pl.ds(c * 16, 16)] for c in range(D // 16)]

        def row_body(r, c2, s=s, tvals=tvals):
            for dr in range(2):
                row = 2 * r + dr
                for c in range(D // 16):
                    sl = pl.ds(c * 16, 16)
                    obuf_v[s, row, sl] = (obuf_v[s, row, sl] * SCALE
                                          + tvals[c])
            return c2

        lax.fori_loop(0, _BQ // 2, row_body, 0)

        pltpu.async_copy(obuf_v.at[s], out_dst(k), osem)
        if k + 2 < _NBLK:
            if k >= 1:
                wait_write(k - 1)
            start_gather(k + 2)

    for k in range(_NBLK - 3, _NBLK):
        wait_write(k)


@functools.partial(jax.jit, static_argnames=())
def kernel(x, W, class_token, pos_encoding):
    xq = x.astype(jnp.int32).T.reshape(L, B // 128, 128)
    pos = pos_encoding[0, : L + 1]
    ct = class_token.reshape(D)

    mesh = plsc.VectorSubcoreMesh(core_axis_name="c", subcore_axis_name="s")
    f = functools.partial(
        pl.kernel,
        mesh=mesh,
        out_type=jax.ShapeDtypeStruct((L + 1, B, D), jnp.float32),
        scratch_types=[
            pltpu.VMEM((D,), jnp.float32),
            pltpu.VMEM((D,), jnp.float32),
            pltpu.VMEM((_NBLK, D), jnp.float32),
            pltpu.VMEM((_NBLK, 2, 128), jnp.int32),
            pltpu.VMEM((_NBUF, _BQ, D), jnp.float32),
            pltpu.SemaphoreType.DMA,
            pltpu.SemaphoreType.DMA,
            pltpu.SemaphoreType.DMA,
        ],
    )(_sc_body)
    out = f(W, xq, pos, ct)
    return jnp.transpose(out, (1, 0, 2))
